# Initial kernel scaffold; baseline (speedup 1.0000x reference)
import jax
import jax.numpy as jnp
from jax.experimental import pallas as pl


def kernel(X, table, W, b):
    raise NotImplementedError("write your pallas kernel here")


# speedup vs baseline: 3.1721x; 3.1721x over previous
import functools

import jax
import jax.numpy as jnp
from jax import lax
from jax.experimental import pallas as pl
from jax.experimental.pallas import tpu as pltpu
from jax.experimental.pallas import tpu_sc as plsc

VOCAB = 100000
EMBED = 64
HIDDEN = 128

NC = 2
NS = 16
NW = NC * NS

CHUNK = 128


def _sc_gather(idx_flat, table):
    n = idx_flat.shape[0]
    assert n % NW == 0
    per_w = n // NW
    assert per_w % CHUNK == 0
    n_chunks = per_w // CHUNK

    mesh = plsc.VectorSubcoreMesh(core_axis_name="c", subcore_axis_name="s")

    @functools.partial(
        pl.kernel,
        mesh=mesh,
        out_type=jax.ShapeDtypeStruct((n, EMBED), jnp.float32),
        compiler_params=pltpu.CompilerParams(use_tc_tiling_on_sc=False),
        scratch_types=[
            pltpu.VMEM((CHUNK,), jnp.int32),
            pltpu.VMEM((CHUNK, EMBED), jnp.float32),
            pltpu.SemaphoreType.DMA,
        ],
    )
    def k(idx_hbm, table_hbm, out_hbm, idx_v, rows_v, sem):
        wid = lax.axis_index("s") * NC + lax.axis_index("c")
        w_base = wid * per_w

        def chunk_body(i, carry):
            base = w_base + i * CHUNK
            pltpu.sync_copy(idx_hbm.at[pl.ds(base, CHUNK)], idx_v)
            pltpu.async_copy(table_hbm.at[idx_v], rows_v, sem).wait()
            pltpu.sync_copy(rows_v, out_hbm.at[pl.ds(base, CHUNK)])
            return carry

        lax.fori_loop(0, n_chunks, chunk_body, 0)

    return k(idx_flat, table)


def _tc_dense_tanh(g, W, b):
    m = g.shape[0]
    bm = 1024
    assert m % bm == 0

    def body(g_ref, w_ref, b_ref, o_ref):
        acc = jnp.dot(g_ref[...], w_ref[...], preferred_element_type=jnp.float32)
        o_ref[...] = jnp.tanh(acc + b_ref[...])

    return pl.pallas_call(
        body,
        grid=(m // bm,),
        in_specs=[
            pl.BlockSpec((bm, 2 * EMBED), lambda i: (i, 0)),
            pl.BlockSpec((2 * EMBED, HIDDEN), lambda i: (0, 0)),
            pl.BlockSpec((1, HIDDEN), lambda i: (0, 0)),
        ],
        out_specs=pl.BlockSpec((bm, HIDDEN), lambda i: (i, 0)),
        out_shape=jax.ShapeDtypeStruct((m, HIDDEN), jnp.float32),
    )(g, W, b.reshape(1, HIDDEN))


def kernel(X, table, W, b):
    c, n, two = X.shape
    idx_flat = X.astype(jnp.int32).reshape(-1)
    gathered = _sc_gather(idx_flat, table)
    g = gathered.reshape(c * n, 2 * EMBED)
    out = _tc_dense_tanh(g, W, b)
    return out.reshape(c, n, HIDDEN)


# speedup vs baseline: 5.4340x; 1.7130x over previous
import functools

import jax
import jax.numpy as jnp
from jax import lax
from jax.experimental import pallas as pl
from jax.experimental.pallas import tpu as pltpu
from jax.experimental.pallas import tpu_sc as plsc

VOCAB = 100000
EMBED = 64
HIDDEN = 128

NC = 2
NS = 16
NW = NC * NS

CHUNK = 128


def _tc_precompute(table, W, b):
    bm = 5000

    def body(x_ref, wa_ref, wb_ref, b_ref, oa_ref, ob_ref):
        x = x_ref[...]
        oa_ref[...] = jnp.dot(x, wa_ref[...], preferred_element_type=jnp.float32) + b_ref[...]
        ob_ref[...] = jnp.dot(x, wb_ref[...], preferred_element_type=jnp.float32)

    return pl.pallas_call(
        body,
        grid=(VOCAB // bm,),
        in_specs=[
            pl.BlockSpec((bm, EMBED), lambda i: (i, 0)),
            pl.BlockSpec((EMBED, HIDDEN), lambda i: (0, 0)),
            pl.BlockSpec((EMBED, HIDDEN), lambda i: (0, 0)),
            pl.BlockSpec((1, HIDDEN), lambda i: (0, 0)),
        ],
        out_specs=[
            pl.BlockSpec((bm, HIDDEN), lambda i: (i, 0)),
            pl.BlockSpec((bm, HIDDEN), lambda i: (i, 0)),
        ],
        out_shape=[
            jax.ShapeDtypeStruct((VOCAB, HIDDEN), jnp.float32),
            jax.ShapeDtypeStruct((VOCAB, HIDDEN), jnp.float32),
        ],
    )(table, W[:EMBED], W[EMBED:], b.reshape(1, HIDDEN))


def _sc_gather_sum(idxa, idxb, ta, tb):
    n = idxa.shape[0]
    assert n % NW == 0
    per_w = n // NW
    assert per_w % CHUNK == 0
    n_chunks = per_w // CHUNK

    mesh = plsc.VectorSubcoreMesh(core_axis_name="c", subcore_axis_name="s")

    @functools.partial(
        pl.kernel,
        mesh=mesh,
        out_type=jax.ShapeDtypeStruct((n, HIDDEN), jnp.float32),
        scratch_types=[
            pltpu.VMEM((CHUNK,), jnp.int32),
            pltpu.VMEM((CHUNK,), jnp.int32),
            pltpu.VMEM((CHUNK, HIDDEN), jnp.float32),
            pltpu.SemaphoreType.DMA,
        ],
    )
    def k(idxa_hbm, idxb_hbm, ta_hbm, tb_hbm, out_hbm, ia_v, ib_v, rows_v, sem):
        wid = lax.axis_index("s") * NC + lax.axis_index("c")
        w_base = wid * per_w

        def chunk_body(i, carry):
            base = w_base + i * CHUNK
            pltpu.sync_copy(idxa_hbm.at[pl.ds(base, CHUNK)], ia_v)
            pltpu.sync_copy(idxb_hbm.at[pl.ds(base, CHUNK)], ib_v)
            pltpu.async_copy(ta_hbm.at[ia_v], rows_v, sem).wait()
            pltpu.async_copy(tb_hbm.at[ib_v], rows_v, sem, add=True).wait()
            pltpu.sync_copy(rows_v, out_hbm.at[pl.ds(base, CHUNK)])
            return carry

        lax.fori_loop(0, n_chunks, chunk_body, 0)

    return k(idxa, idxb, ta, tb)


def _tc_tanh(x):
    m = x.shape[0]
    bm = 2048
    assert m % bm == 0

    def body(x_ref, o_ref):
        o_ref[...] = jnp.tanh(x_ref[...])

    return pl.pallas_call(
        body,
        grid=(m // bm,),
        in_specs=[pl.BlockSpec((bm, HIDDEN), lambda i: (i, 0))],
        out_specs=pl.BlockSpec((bm, HIDDEN), lambda i: (i, 0)),
        out_shape=jax.ShapeDtypeStruct((m, HIDDEN), jnp.float32),
    )(x)


def kernel(X, table, W, b):
    c, n, _ = X.shape
    xi = X.astype(jnp.int32)
    idxa = xi[:, :, 0].reshape(-1)
    idxb = xi[:, :, 1].reshape(-1)
    ta, tb = _tc_precompute(table, W, b)
    s = _sc_gather_sum(idxa, idxb, ta, tb)
    out = _tc_tanh(s)
    return out.reshape(c, n, HIDDEN)


# speedup vs baseline: 7.8964x; 1.4532x over previous
import functools

import jax
import jax.numpy as jnp
from jax import lax
from jax.experimental import pallas as pl
from jax.experimental.pallas import tpu as pltpu
from jax.experimental.pallas import tpu_sc as plsc

VOCAB = 100000
EMBED = 64
HIDDEN = 128

NC = 2
NS = 16
NW = NC * NS

CHUNK = 128


def _tc_precompute(table, W, b):
    bm = 5000

    def body(x_ref, wa_ref, wb_ref, b_ref, oa_ref, ob_ref):
        x = x_ref[...]
        oa_ref[...] = jnp.dot(x, wa_ref[...], preferred_element_type=jnp.float32) + b_ref[...]
        ob_ref[...] = jnp.dot(x, wb_ref[...], preferred_element_type=jnp.float32)

    return pl.pallas_call(
        body,
        grid=(VOCAB // bm,),
        in_specs=[
            pl.BlockSpec((bm, EMBED), lambda i: (i, 0)),
            pl.BlockSpec((EMBED, HIDDEN), lambda i: (0, 0)),
            pl.BlockSpec((EMBED, HIDDEN), lambda i: (0, 0)),
            pl.BlockSpec((1, HIDDEN), lambda i: (0, 0)),
        ],
        out_specs=[
            pl.BlockSpec((bm, HIDDEN), lambda i: (i, 0)),
            pl.BlockSpec((bm, HIDDEN), lambda i: (i, 0)),
        ],
        out_shape=[
            jax.ShapeDtypeStruct((VOCAB, HIDDEN), jnp.float32),
            jax.ShapeDtypeStruct((VOCAB, HIDDEN), jnp.float32),
        ],
    )(table, W[:EMBED], W[EMBED:], b.reshape(1, HIDDEN))


SLOTS = 4


def _sc_gather_sum(idxa, idxb, ta, tb):
    n = idxa.shape[0]
    assert n % NW == 0
    per_w = n // NW
    assert per_w % CHUNK == 0
    n_chunks = per_w // CHUNK
    idxa2 = idxa.reshape(-1, CHUNK)
    idxb2 = idxb.reshape(-1, CHUNK)

    mesh = plsc.VectorSubcoreMesh(core_axis_name="c", subcore_axis_name="s")

    @functools.partial(
        pl.kernel,
        mesh=mesh,
        out_type=jax.ShapeDtypeStruct((n, HIDDEN), jnp.float32),
        scratch_types=[
            pltpu.VMEM((SLOTS, CHUNK), jnp.int32),
            pltpu.VMEM((SLOTS, CHUNK), jnp.int32),
            pltpu.VMEM((SLOTS, CHUNK, HIDDEN), jnp.float32),
            pltpu.SemaphoreType.DMA((SLOTS,)),
            pltpu.SemaphoreType.DMA((SLOTS,)),
            pltpu.SemaphoreType.DMA((SLOTS,)),
            pltpu.SemaphoreType.DMA((SLOTS,)),
        ],
    )
    def k(idxa_hbm, idxb_hbm, ta_hbm, tb_hbm, out_hbm,
          ia, ib, rows, sem_idx, sem_g, sem_ad, sem_st):
        wid = lax.axis_index("s") * NC + lax.axis_index("c")
        blk0 = wid * n_chunks
        row0 = wid * per_w

        def slot(i):
            return lax.rem(i + 4 * n_chunks, SLOTS)

        def idx_copies(i):
            s = slot(i)
            return (pltpu.make_async_copy(idxa_hbm.at[blk0 + i], ia.at[s], sem_idx.at[s]),
                    pltpu.make_async_copy(idxb_hbm.at[blk0 + i], ib.at[s], sem_idx.at[s]))

        def ga_copy(i):
            s = slot(i)
            return pltpu.make_async_copy(ta_hbm.at[ia.at[s]], rows.at[s], sem_g.at[s])

        def st_copy(i):
            s = slot(i)
            return pltpu.make_async_copy(
                rows.at[s], out_hbm.at[pl.ds(row0 + i * CHUNK, CHUNK)], sem_st.at[s])

        def issue_gadd(i):
            s = slot(i)
            pltpu.async_copy(tb_hbm.at[ib.at[s]], rows.at[s], sem_ad.at[s], add=True)

        def wait_gadd(i):
            s = slot(i)
            pltpu.make_async_copy(tb_hbm.at[ib.at[s]], rows.at[s], sem_ad.at[s]).wait()

        for d in idx_copies(0):
            d.start()
        for d in idx_copies(1):
            d.start()
        for d in idx_copies(0):
            d.wait()
        ga_copy(0).start()

        def body(i, carry):
            @pl.when(i + 2 < n_chunks)
            def _():
                for d in idx_copies(i + 2):
                    d.start()
            ga_copy(i).wait()
            issue_gadd(i)

            @pl.when(i + 1 < n_chunks)
            def _():
                for d in idx_copies(i + 1):
                    d.wait()

                @pl.when(i - 3 >= 0)
                def _():
                    st_copy(i - 3).wait()
                ga_copy(i + 1).start()

            wait_gadd(i)
            st_copy(i).start()
            return carry

        lax.fori_loop(0, n_chunks, body, 0)
        for i in range(n_chunks - SLOTS, n_chunks):
            st_copy(i).wait()

    return k(idxa2, idxb2, ta, tb)


def _tc_tanh(x):
    m = x.shape[0]
    bm = 2048
    assert m % bm == 0

    def body(x_ref, o_ref):
        o_ref[...] = jnp.tanh(x_ref[...])

    return pl.pallas_call(
        body,
        grid=(m // bm,),
        in_specs=[pl.BlockSpec((bm, HIDDEN), lambda i: (i, 0))],
        out_specs=pl.BlockSpec((bm, HIDDEN), lambda i: (i, 0)),
        out_shape=jax.ShapeDtypeStruct((m, HIDDEN), jnp.float32),
    )(x)


def kernel(X, table, W, b):
    c, n, _ = X.shape
    xi = X.astype(jnp.int32)
    idxa = xi[:, :, 0].reshape(-1)
    idxb = xi[:, :, 1].reshape(-1)
    ta, tb = _tc_precompute(table, W, b)
    s = _sc_gather_sum(idxa, idxb, ta, tb)
    out = _tc_tanh(s)
    return out.reshape(c, n, HIDDEN)


# speedup vs baseline: 9.1162x; 1.1545x over previous
import functools

import jax
import jax.numpy as jnp
from jax import lax
from jax.experimental import pallas as pl
from jax.experimental.pallas import tpu as pltpu
from jax.experimental.pallas import tpu_sc as plsc

VOCAB = 100000
EMBED = 64
HIDDEN = 128

NC = 2
NS = 16
NW = NC * NS

CHUNK = 128


def _tc_precompute(table, W, b):
    bm = 5000

    def body(x_ref, wa_ref, wb_ref, b_ref, oa_ref, ob_ref):
        x = x_ref[...]
        oa_ref[...] = jnp.dot(x, wa_ref[...], preferred_element_type=jnp.float32) + b_ref[...]
        ob_ref[...] = jnp.dot(x, wb_ref[...], preferred_element_type=jnp.float32)

    return pl.pallas_call(
        body,
        grid=(VOCAB // bm,),
        in_specs=[
            pl.BlockSpec((bm, EMBED), lambda i: (i, 0)),
            pl.BlockSpec((EMBED, HIDDEN), lambda i: (0, 0)),
            pl.BlockSpec((EMBED, HIDDEN), lambda i: (0, 0)),
            pl.BlockSpec((1, HIDDEN), lambda i: (0, 0)),
        ],
        out_specs=[
            pl.BlockSpec((bm, HIDDEN), lambda i: (i, 0)),
            pl.BlockSpec((bm, HIDDEN), lambda i: (i, 0)),
        ],
        out_shape=[
            jax.ShapeDtypeStruct((VOCAB, HIDDEN), jnp.float32),
            jax.ShapeDtypeStruct((VOCAB, HIDDEN), jnp.float32),
        ],
    )(table, W[:EMBED], W[EMBED:], b.reshape(1, HIDDEN))


SLOTS = 4


def _sc_gather_sum(idxa, idxb, ta, tb):
    n = idxa.shape[0]
    assert n % NW == 0
    per_w = n // NW
    assert per_w % CHUNK == 0
    n_chunks = per_w // CHUNK
    idxa2 = idxa.reshape(-1, CHUNK)
    idxb2 = idxb.reshape(-1, CHUNK)

    mesh = plsc.VectorSubcoreMesh(core_axis_name="c", subcore_axis_name="s")

    @functools.partial(
        pl.kernel,
        mesh=mesh,
        out_type=jax.ShapeDtypeStruct((n, HIDDEN), jnp.float32),
        scratch_types=[
            pltpu.VMEM((SLOTS, CHUNK), jnp.int32),
            pltpu.VMEM((SLOTS, CHUNK), jnp.int32),
            pltpu.VMEM((SLOTS, CHUNK, HIDDEN), jnp.float32),
            pltpu.SemaphoreType.DMA((SLOTS,)),
            pltpu.SemaphoreType.DMA((SLOTS,)),
            pltpu.SemaphoreType.DMA((SLOTS,)),
            pltpu.SemaphoreType.DMA((SLOTS,)),
        ],
    )
    def k(idxa_hbm, idxb_hbm, ta_hbm, tb_hbm, out_hbm,
          ia, ib, rows, sem_idx, sem_g, sem_ad, sem_st):
        wid = lax.axis_index("s") * NC + lax.axis_index("c")
        blk0 = wid * n_chunks
        row0 = wid * per_w

        def slot(i):
            return lax.rem(i + 4 * n_chunks, SLOTS)

        def idx_copies(i):
            s = slot(i)
            return (pltpu.make_async_copy(idxa_hbm.at[blk0 + i], ia.at[s], sem_idx.at[s]),
                    pltpu.make_async_copy(idxb_hbm.at[blk0 + i], ib.at[s], sem_idx.at[s]))

        def ga_copy(i):
            s = slot(i)
            return pltpu.make_async_copy(ta_hbm.at[ia.at[s]], rows.at[s], sem_g.at[s])

        def st_copy(i):
            s = slot(i)
            return pltpu.make_async_copy(
                rows.at[s], out_hbm.at[pl.ds(row0 + i * CHUNK, CHUNK)], sem_st.at[s])

        def issue_gadd(i):
            s = slot(i)
            pltpu.async_copy(tb_hbm.at[ib.at[s]], rows.at[s], sem_ad.at[s], add=True)

        def wait_gadd(i):
            s = slot(i)
            pltpu.make_async_copy(tb_hbm.at[ib.at[s]], rows.at[s], sem_ad.at[s]).wait()

        for d in idx_copies(0):
            d.start()
        for d in idx_copies(1):
            d.start()
        for d in idx_copies(0):
            d.wait()
        ga_copy(0).start()

        def body(i, carry):
            @pl.when(i + 2 < n_chunks)
            def _():
                for d in idx_copies(i + 2):
                    d.start()
            ga_copy(i).wait()
            issue_gadd(i)

            @pl.when(i + 1 < n_chunks)
            def _():
                for d in idx_copies(i + 1):
                    d.wait()

                @pl.when(i - 3 >= 0)
                def _():
                    st_copy(i - 3).wait()
                ga_copy(i + 1).start()

            wait_gadd(i)
            st_copy(i).start()
            return carry

        lax.fori_loop(0, n_chunks, body, 0)
        for i in range(n_chunks - SLOTS, n_chunks):
            st_copy(i).wait()

    return k(idxa2, idxb2, ta, tb)


NSLICES = 4
TANH_BM = 2048


def _tc_tanh_slice(x, buf, m_total, sl):
    m = x.shape[0]
    assert m % TANH_BM == 0
    nb = m // TANH_BM

    if buf is None:
        def body(x_ref, o_ref):
            o_ref[...] = jnp.tanh(x_ref[...])
        in_specs = [pl.BlockSpec((TANH_BM, HIDDEN), lambda i: (i, 0))]
        args = (x,)
        aliases = {}
    else:
        def body(x_ref, b_ref, o_ref):
            o_ref[...] = jnp.tanh(x_ref[...])
        in_specs = [
            pl.BlockSpec((TANH_BM, HIDDEN), lambda i: (i, 0)),
            pl.BlockSpec(memory_space=pl.ANY),
        ]
        args = (x, buf)
        aliases = {1: 0}

    return pl.pallas_call(
        body,
        grid=(nb,),
        in_specs=in_specs,
        out_specs=pl.BlockSpec((TANH_BM, HIDDEN), lambda i: (sl * nb + i, 0)),
        out_shape=jax.ShapeDtypeStruct((m_total, HIDDEN), jnp.float32),
        input_output_aliases=aliases,
    )(*args)


def kernel(X, table, W, b):
    c, n, _ = X.shape
    m = c * n
    xi = X.astype(jnp.int32)
    idxa = xi[:, :, 0].reshape(-1)
    idxb = xi[:, :, 1].reshape(-1)
    ta, tb = _tc_precompute(table, W, b)

    sl_m = m // NSLICES
    sums = [
        _sc_gather_sum(idxa[s * sl_m:(s + 1) * sl_m],
                       idxb[s * sl_m:(s + 1) * sl_m], ta, tb)
        for s in range(NSLICES)
    ]
    buf = None
    for s in range(NSLICES):
        buf = _tc_tanh_slice(sums[s], buf, m, s)
    return buf.reshape(c, n, HIDDEN)
